# Initial kernel scaffold; baseline (speedup 1.0000x reference)
import jax
import jax.numpy as jnp
from jax.experimental import pallas as pl


def kernel(article, article_length, prev_input, prev_input_length, table, W_enc, W_dec, W_attn, w_gen, w_agent, W_out):
    raise NotImplementedError("write your pallas kernel here")


# speedup vs baseline: 2.8101x; 2.8101x over previous
import functools

import jax
import jax.numpy as jnp
from jax import lax
from jax.experimental import pallas as pl
from jax.experimental.pallas import tpu as pltpu
from jax.experimental.pallas import tpu_sc as plsc

V = 50000
EXT = 51000
D = 128
H = 256
S = 400
B = 4
A = 3
T = 32
BT = B * T
VT = 1024
NT = -(-EXT // VT)
ZP = 51200

NC = 2
NS = 16
NW = NC * NS

NG = 5120
GPW = NG // NW


def _gather_sc(table, idx_all):
    mesh = plsc.VectorSubcoreMesh(core_axis_name="c", subcore_axis_name="s")

    @functools.partial(
        pl.kernel,
        out_type=jax.ShapeDtypeStruct((NG, D), jnp.float32),
        mesh=mesh,
        scratch_types=[
            pltpu.VMEM((GPW // 2,), jnp.int32),
            pltpu.VMEM((GPW // 2,), jnp.int32),
            pltpu.VMEM((GPW, D), jnp.float32),
            pltpu.SemaphoreType.DMA,
            pltpu.SemaphoreType.DMA,
        ],
    )
    def k(table_hbm, idx_hbm, out_hbm, idx_a, idx_b, rows_v, sem_a, sem_b):
        wid = lax.axis_index("s") * NC + lax.axis_index("c")
        base = wid * GPW
        half = GPW // 2
        pltpu.sync_copy(idx_hbm.at[pl.ds(base, half)], idx_a)
        pltpu.sync_copy(idx_hbm.at[pl.ds(base + half, half)], idx_b)
        c1 = pltpu.async_copy(table_hbm.at[idx_a], rows_v.at[pl.ds(0, half)], sem_a)
        c2 = pltpu.async_copy(table_hbm.at[idx_b], rows_v.at[pl.ds(half, half)], sem_b)
        c1.wait()
        c2.wait()
        pltpu.sync_copy(rows_v, out_hbm.at[pl.ds(base, GPW)])

    return k(table, idx_all)


def _attention_tc(emb_art, emb_prev2d, W_enc, W_dec, W_attn, wgen_row,
                  wagent_row, lengths, prev_len):
    def body(emb_art_ref, emb_prev_ref, wenc_ref, wdec_ref, wattn_ref,
             wgen_ref, wagent_ref, len_ref, plen_ref,
             dec_ref, gmix_ref, cw_ref):
        dec2d = jnp.tanh(jnp.dot(emb_prev_ref[...], wdec_ref[...],
                                 preferred_element_type=jnp.float32))
        proj = jnp.dot(dec2d, wattn_ref[...], preferred_element_type=jnp.float32)
        dec_ref[...] = dec2d
        for b in range(B):
            proj_b = proj[b * T:(b + 1) * T]
            dec_b = dec2d[b * T:(b + 1) * T]
            attns, ges, gas = [], [], []
            for a in range(A):
                enc = jnp.tanh(jnp.dot(emb_art_ref[b, a], wenc_ref[...],
                                       preferred_element_type=jnp.float32))
                sc = lax.dot_general(proj_b, enc, (((1,), (1,)), ((), ())),
                                     preferred_element_type=jnp.float32)
                iota_s = lax.broadcasted_iota(jnp.int32, (T, S), 1)
                sc = jnp.where(iota_s < len_ref[b * A + a], sc, -1e9)
                m = jnp.max(sc, axis=1, keepdims=True)
                e = jnp.exp(sc - m)
                attn = e / jnp.sum(e, axis=1, keepdims=True)
                ctx = jnp.dot(attn, enc, preferred_element_type=jnp.float32)
                comb = ctx + dec_b
                ge = 1.0 / (1.0 + jnp.exp(-jnp.sum(comb * wgen_ref[...],
                                                   axis=1, keepdims=True)))
                ga = jnp.sum(comb * wagent_ref[...], axis=1, keepdims=True)
                attns.append(attn)
                ges.append(ge)
                gas.append(ga)
            m = jnp.maximum(jnp.maximum(gas[0], gas[1]), gas[2])
            es = [jnp.exp(g - m) for g in gas]
            den = es[0] + es[1] + es[2]
            iota_t = lax.broadcasted_iota(jnp.int32, (T, 1), 0)
            tgt = (iota_t < plen_ref[b]).astype(jnp.float32)
            aas = [e / den for e in es]
            gmix_ref[b * T:(b + 1) * T, :] = (
                aas[0] * ges[0] + aas[1] * ges[1] + aas[2] * ges[2]) * tgt
            for a in range(A):
                cw_ref[a, b * T:(b + 1) * T, :] = (
                    aas[a] * (1.0 - ges[a]) * tgt) * attns[a]

    return pl.pallas_call(
        body,
        out_shape=[
            jax.ShapeDtypeStruct((BT, H), jnp.float32),
            jax.ShapeDtypeStruct((BT, 1), jnp.float32),
            jax.ShapeDtypeStruct((A, BT, S), jnp.float32),
        ],
        in_specs=[pl.BlockSpec(memory_space=pltpu.VMEM)] * 7
        + [pl.BlockSpec(memory_space=pltpu.SMEM)] * 2,
        out_specs=[pl.BlockSpec(memory_space=pltpu.VMEM)] * 3,
    )(emb_art, emb_prev2d, W_enc, W_dec, W_attn, wgen_row, wagent_row,
      lengths, prev_len)


def _scatter_sc(cw, art_bas):
    mesh = plsc.VectorSubcoreMesh(core_axis_name="c", subcore_axis_name="s")

    @functools.partial(
        pl.kernel,
        out_type=jax.ShapeDtypeStruct((BT * EXT,), jnp.float32),
        mesh=mesh,
        compiler_params=pltpu.CompilerParams(needs_layout_passes=False),
        scratch_types=[
            pltpu.VMEM((A * S,), jnp.int32),
            pltpu.VMEM((A * S,), jnp.float32),
            pltpu.VMEM((ZP,), jnp.float32),
        ],
    )
    def k(cw_hbm, art_hbm, out_hbm, idx_v, val_v, rowbuf):
        wid = lax.axis_index("s") * NC + lax.axis_index("c")
        b = wid // (NW // B)
        pltpu.sync_copy(art_hbm.at[pl.ds(b * A * S, A * S)], idx_v)
        zero16 = jnp.zeros((16,), jnp.float32)
        for r in range(BT // NW):
            row = wid * (BT // NW) + r

            def zbody(i, carry):
                base = i * 128
                for k2 in range(8):
                    rowbuf[pl.ds(base + k2 * 16, 16)] = zero16
                return carry

            lax.fori_loop(0, ZP // 128, zbody, 0)
            for a in range(A):
                pltpu.sync_copy(
                    cw_hbm.at[pl.ds((a * BT + row) * S, S)],
                    val_v.at[pl.ds(a * S, S)])
            for c in range(A * S // 16):
                plsc.addupdate_scatter(
                    rowbuf,
                    [idx_v[pl.ds(c * 16, 16)]],
                    val_v[pl.ds(c * 16, 16)])
            pltpu.sync_copy(rowbuf.at[pl.ds(0, EXT)], out_hbm.at[pl.ds(row * EXT, EXT)])

    return k(cw.reshape(-1), art_bas.reshape(-1))


def _vocab_tc(dec2d, W_out, gmix, scat):
    def body(dec_ref, wout_ref, gmix_ref, scat_ref, out_ref, acc_ref):
        p = pl.program_id(0)
        j = pl.program_id(1)

        @pl.when(jnp.logical_and(p == 0, j == 0))
        def _():
            acc_ref[...] = jnp.zeros_like(acc_ref)

        l = jnp.dot(dec_ref[...], wout_ref[...],
                    preferred_element_type=jnp.float32)
        gcol = j * VT + lax.broadcasted_iota(jnp.int32, (BT, VT), 1)
        e = jnp.where(gcol < V, jnp.exp(l), 0.0)

        @pl.when(p == 0)
        def _():
            acc_ref[...] = acc_ref[...] + jnp.sum(e, axis=1, keepdims=True)

        @pl.when(p == 1)
        def _():
            out_ref[...] = gmix_ref[...] * e / acc_ref[...] + scat_ref[...]

    return pl.pallas_call(
        body,
        grid=(2, NT),
        in_specs=[
            pl.BlockSpec((BT, H), lambda p, j: (0, 0)),
            pl.BlockSpec((H, VT), lambda p, j: (0, jnp.minimum(j, (V - 1) // VT))),
            pl.BlockSpec((BT, 1), lambda p, j: (0, 0)),
            pl.BlockSpec((BT, VT), lambda p, j: (0, j * p)),
        ],
        out_specs=pl.BlockSpec((BT, VT), lambda p, j: (0, j * p)),
        out_shape=jax.ShapeDtypeStruct((BT, EXT), jnp.float32),
        scratch_shapes=[pltpu.VMEM((BT, 1), jnp.float32)],
    )(dec2d, W_out, gmix, scat)


def kernel(article, article_length, prev_input, prev_input_length, table,
           W_enc, W_dec, W_attn, w_gen, w_agent, W_out):
    art_bas = article.transpose(1, 2, 0).astype(jnp.int32)
    idx_art = art_bas.reshape(-1)
    idx_prev = prev_input.transpose(1, 0).reshape(-1)
    idx_all = jnp.concatenate(
        [idx_art, idx_prev,
         jnp.zeros((NG - idx_art.size - idx_prev.size,), jnp.int32)])
    emb_all = _gather_sc(table, idx_all)
    emb_art = emb_all[:B * A * S].reshape(B, A, S, D)
    emb_prev2d = emb_all[B * A * S:B * A * S + BT]

    lengths = jnp.maximum(article_length, 1).astype(jnp.int32)
    prev_len = jnp.maximum(prev_input_length, 1).astype(jnp.int32)

    dec2d, gmix, cw = _attention_tc(
        emb_art, emb_prev2d, W_enc, W_dec, W_attn,
        w_gen.reshape(1, H), w_agent.reshape(1, H), lengths, prev_len)

    scat = _scatter_sc(cw, art_bas).reshape(BT, EXT)
    out = _vocab_tc(dec2d, W_out, gmix, scat)
    return out.reshape(B, T, EXT)


# speedup vs baseline: 2.9032x; 1.0331x over previous
import functools

import jax
import jax.numpy as jnp
from jax import lax
from jax.experimental import pallas as pl
from jax.experimental.pallas import tpu as pltpu
from jax.experimental.pallas import tpu_sc as plsc

V = 50000
EXT = 51000
D = 128
H = 256
S = 400
B = 4
A = 3
T = 32
BT = B * T
VT = 1024
NT = -(-EXT // VT)
ZP = 51200

NC = 2
NS = 16
NW = NC * NS

NG = 5120
GPW = NG // NW


def _gather_sc(table, idx_all):
    mesh = plsc.VectorSubcoreMesh(core_axis_name="c", subcore_axis_name="s")

    @functools.partial(
        pl.kernel,
        out_type=jax.ShapeDtypeStruct((NG, D), jnp.float32),
        mesh=mesh,
        scratch_types=[
            pltpu.VMEM((GPW // 2,), jnp.int32),
            pltpu.VMEM((GPW // 2,), jnp.int32),
            pltpu.VMEM((GPW, D), jnp.float32),
            pltpu.SemaphoreType.DMA,
            pltpu.SemaphoreType.DMA,
        ],
    )
    def k(table_hbm, idx_hbm, out_hbm, idx_a, idx_b, rows_v, sem_a, sem_b):
        wid = lax.axis_index("s") * NC + lax.axis_index("c")
        base = wid * GPW
        half = GPW // 2
        pltpu.sync_copy(idx_hbm.at[pl.ds(base, half)], idx_a)
        pltpu.sync_copy(idx_hbm.at[pl.ds(base + half, half)], idx_b)
        c1 = pltpu.async_copy(table_hbm.at[idx_a], rows_v.at[pl.ds(0, half)], sem_a)
        c2 = pltpu.async_copy(table_hbm.at[idx_b], rows_v.at[pl.ds(half, half)], sem_b)
        c1.wait()
        c2.wait()
        pltpu.sync_copy(rows_v, out_hbm.at[pl.ds(base, GPW)])

    return k(table, idx_all)


def _attention_tc(emb_art, emb_prev2d, W_enc, W_dec, W_attn, wgen_row,
                  wagent_row, lengths, prev_len):
    def body(emb_art_ref, emb_prev_ref, wenc_ref, wdec_ref, wattn_ref,
             wgen_ref, wagent_ref, len_ref, plen_ref,
             dec_ref, gmix_ref, cw_ref):
        dec2d = jnp.tanh(jnp.dot(emb_prev_ref[...], wdec_ref[...],
                                 preferred_element_type=jnp.float32))
        proj = jnp.dot(dec2d, wattn_ref[...], preferred_element_type=jnp.float32)
        dec_ref[...] = dec2d
        for b in range(B):
            proj_b = proj[b * T:(b + 1) * T]
            dec_b = dec2d[b * T:(b + 1) * T]
            attns, ges, gas = [], [], []
            for a in range(A):
                enc = jnp.tanh(jnp.dot(emb_art_ref[b, a], wenc_ref[...],
                                       preferred_element_type=jnp.float32))
                sc = lax.dot_general(proj_b, enc, (((1,), (1,)), ((), ())),
                                     preferred_element_type=jnp.float32)
                iota_s = lax.broadcasted_iota(jnp.int32, (T, S), 1)
                sc = jnp.where(iota_s < len_ref[b * A + a], sc, -1e9)
                m = jnp.max(sc, axis=1, keepdims=True)
                e = jnp.exp(sc - m)
                attn = e / jnp.sum(e, axis=1, keepdims=True)
                ctx = jnp.dot(attn, enc, preferred_element_type=jnp.float32)
                comb = ctx + dec_b
                ge = 1.0 / (1.0 + jnp.exp(-jnp.sum(comb * wgen_ref[...],
                                                   axis=1, keepdims=True)))
                ga = jnp.sum(comb * wagent_ref[...], axis=1, keepdims=True)
                attns.append(attn)
                ges.append(ge)
                gas.append(ga)
            m = jnp.maximum(jnp.maximum(gas[0], gas[1]), gas[2])
            es = [jnp.exp(g - m) for g in gas]
            den = es[0] + es[1] + es[2]
            iota_t = lax.broadcasted_iota(jnp.int32, (T, 1), 0)
            tgt = (iota_t < plen_ref[b]).astype(jnp.float32)
            aas = [e / den for e in es]
            gmix_ref[b * T:(b + 1) * T, :] = (
                aas[0] * ges[0] + aas[1] * ges[1] + aas[2] * ges[2]) * tgt
            for a in range(A):
                cw_ref[a, b * T:(b + 1) * T, :] = (
                    aas[a] * (1.0 - ges[a]) * tgt) * attns[a]

    return pl.pallas_call(
        body,
        out_shape=[
            jax.ShapeDtypeStruct((BT, H), jnp.float32),
            jax.ShapeDtypeStruct((BT, 1), jnp.float32),
            jax.ShapeDtypeStruct((A, BT, S), jnp.float32),
        ],
        in_specs=[pl.BlockSpec(memory_space=pltpu.VMEM)] * 7
        + [pl.BlockSpec(memory_space=pltpu.SMEM)] * 2,
        out_specs=[pl.BlockSpec(memory_space=pltpu.VMEM)] * 3,
    )(emb_art, emb_prev2d, W_enc, W_dec, W_attn, wgen_row, wagent_row,
      lengths, prev_len)


def _scatter_sc(cw, art_bas):
    mesh = plsc.VectorSubcoreMesh(core_axis_name="c", subcore_axis_name="s")

    @functools.partial(
        pl.kernel,
        out_type=jax.ShapeDtypeStruct((BT * EXT,), jnp.float32),
        mesh=mesh,
        compiler_params=pltpu.CompilerParams(needs_layout_passes=False),
        scratch_types=[
            pltpu.VMEM((A * S,), jnp.int32),
            pltpu.VMEM((A * S,), jnp.float32),
            pltpu.VMEM((ZP,), jnp.float32),
        ],
    )
    def k(cw_hbm, art_hbm, out_hbm, idx_v, val_v, rowbuf):
        wid = lax.axis_index("s") * NC + lax.axis_index("c")
        b = wid // (NW // B)
        pltpu.sync_copy(art_hbm.at[pl.ds(b * A * S, A * S)], idx_v)
        zero16 = jnp.zeros((16,), jnp.float32)
        for r in range(BT // NW):
            row = wid * (BT // NW) + r

            def zbody(i, carry):
                base = i * 128
                for k2 in range(8):
                    rowbuf[pl.ds(base + k2 * 16, 16)] = zero16
                return carry

            lax.fori_loop(0, ZP // 128, zbody, 0)
            for a in range(A):
                pltpu.sync_copy(
                    cw_hbm.at[pl.ds((a * BT + row) * S, S)],
                    val_v.at[pl.ds(a * S, S)])
            for c in range(A * S // 16):
                plsc.addupdate_scatter(
                    rowbuf,
                    [idx_v[pl.ds(c * 16, 16)]],
                    val_v[pl.ds(c * 16, 16)])
            pltpu.sync_copy(rowbuf.at[pl.ds(0, EXT)], out_hbm.at[pl.ds(row * EXT, EXT)])

    return k(cw.reshape(-1), art_bas.reshape(-1))


def _vocab_tc(dec_bf, W_out_bf, gmix, scat):
    def body(dec_ref, wout_ref, gmix_ref, scat_ref, out_ref, e_scr, acc_ref):
        p = pl.program_id(0)
        j = pl.program_id(1)

        @pl.when(jnp.logical_and(p == 0, j == 0))
        def _():
            acc_ref[...] = jnp.zeros_like(acc_ref)

        @pl.when(p == 0)
        def _():
            l = jnp.dot(dec_ref[...], wout_ref[...],
                        preferred_element_type=jnp.float32)
            gcol = j * VT + lax.broadcasted_iota(jnp.int32, (BT, VT), 1)
            e = jnp.where(gcol < V, jnp.exp(l), 0.0)
            e_scr[:, pl.ds(j * VT, VT)] = e
            acc_ref[...] = acc_ref[...] + jnp.sum(e, axis=1, keepdims=True)

        @pl.when(p == 1)
        def _():
            out_ref[...] = (gmix_ref[...] * e_scr[:, pl.ds(j * VT, VT)]
                            / acc_ref[...] + scat_ref[...])

    return pl.pallas_call(
        body,
        grid=(2, NT),
        in_specs=[
            pl.BlockSpec((BT, H), lambda p, j: (0, 0)),
            pl.BlockSpec(
                (H, VT),
                lambda p, j: (0, jnp.minimum(j, (V - 1) // VT) * (1 - p))),
            pl.BlockSpec((BT, 1), lambda p, j: (0, 0)),
            pl.BlockSpec((BT, VT), lambda p, j: (0, j * p)),
        ],
        out_specs=pl.BlockSpec((BT, VT), lambda p, j: (0, j * p)),
        out_shape=jax.ShapeDtypeStruct((BT, EXT), jnp.float32),
        scratch_shapes=[pltpu.VMEM((BT, NT * VT), jnp.float32),
                        pltpu.VMEM((BT, 1), jnp.float32)],
    )(dec_bf, W_out_bf, gmix, scat)


def kernel(article, article_length, prev_input, prev_input_length, table,
           W_enc, W_dec, W_attn, w_gen, w_agent, W_out):
    art_bas = article.transpose(1, 2, 0).astype(jnp.int32)
    idx_art = art_bas.reshape(-1)
    idx_prev = prev_input.transpose(1, 0).reshape(-1)
    idx_all = jnp.concatenate(
        [idx_art, idx_prev,
         jnp.zeros((NG - idx_art.size - idx_prev.size,), jnp.int32)])
    emb_all = _gather_sc(table, idx_all)
    emb_art = emb_all[:B * A * S].reshape(B, A, S, D)
    emb_prev2d = emb_all[B * A * S:B * A * S + BT]

    lengths = jnp.maximum(article_length, 1).astype(jnp.int32)
    prev_len = jnp.maximum(prev_input_length, 1).astype(jnp.int32)

    dec2d, gmix, cw = _attention_tc(
        emb_art, emb_prev2d, W_enc, W_dec, W_attn,
        w_gen.reshape(1, H), w_agent.reshape(1, H), lengths, prev_len)

    scat = _scatter_sc(cw, art_bas).reshape(BT, EXT)
    out = _vocab_tc(dec2d.astype(jnp.bfloat16), W_out.astype(jnp.bfloat16),
                    gmix, scat)
    return out.reshape(B, T, EXT)


# speedup vs baseline: 5.4038x; 1.8613x over previous
import functools

import jax
import jax.numpy as jnp
from jax import lax
from jax.experimental import pallas as pl
from jax.experimental.pallas import tpu as pltpu
from jax.experimental.pallas import tpu_sc as plsc

V = 50000
EXT = 51000
D = 128
H = 256
S = 400
B = 4
A = 3
T = 32
BT = B * T
VT = 1024
NT = -(-EXT // VT)
ZP = 51200

NC = 2
NS = 16
NW = NC * NS

NG = 5120
GPW = NG // NW


def _gather_sc(table, idx_all):
    mesh = plsc.VectorSubcoreMesh(core_axis_name="c", subcore_axis_name="s")

    @functools.partial(
        pl.kernel,
        out_type=jax.ShapeDtypeStruct((NG, D), jnp.float32),
        mesh=mesh,
        scratch_types=[
            pltpu.VMEM((GPW // 2,), jnp.int32),
            pltpu.VMEM((GPW // 2,), jnp.int32),
            pltpu.VMEM((GPW, D), jnp.float32),
            pltpu.SemaphoreType.DMA,
            pltpu.SemaphoreType.DMA,
        ],
    )
    def k(table_hbm, idx_hbm, out_hbm, idx_a, idx_b, rows_v, sem_a, sem_b):
        wid = lax.axis_index("s") * NC + lax.axis_index("c")
        base = wid * GPW
        half = GPW // 2
        pltpu.sync_copy(idx_hbm.at[pl.ds(base, half)], idx_a)
        pltpu.sync_copy(idx_hbm.at[pl.ds(base + half, half)], idx_b)
        c1 = pltpu.async_copy(table_hbm.at[idx_a], rows_v.at[pl.ds(0, half)], sem_a)
        c2 = pltpu.async_copy(table_hbm.at[idx_b], rows_v.at[pl.ds(half, half)], sem_b)
        c1.wait()
        c2.wait()
        pltpu.sync_copy(rows_v, out_hbm.at[pl.ds(base, GPW)])

    return k(table, idx_all)


def _attention_tc(emb_art, emb_prev2d, W_enc, W_dec, W_attn, wgen_row,
                  wagent_row, lengths, prev_len):
    def body(emb_art_ref, emb_prev_ref, wenc_ref, wdec_ref, wattn_ref,
             wgen_ref, wagent_ref, len_ref, plen_ref,
             dec_ref, gmix_ref, cw_ref):
        dec2d = jnp.tanh(jnp.dot(emb_prev_ref[...], wdec_ref[...],
                                 preferred_element_type=jnp.float32))
        proj = jnp.dot(dec2d, wattn_ref[...], preferred_element_type=jnp.float32)
        dec_ref[...] = dec2d
        for b in range(B):
            proj_b = proj[b * T:(b + 1) * T]
            dec_b = dec2d[b * T:(b + 1) * T]
            attns, ges, gas = [], [], []
            for a in range(A):
                enc = jnp.tanh(jnp.dot(emb_art_ref[b, a], wenc_ref[...],
                                       preferred_element_type=jnp.float32))
                sc = lax.dot_general(proj_b, enc, (((1,), (1,)), ((), ())),
                                     preferred_element_type=jnp.float32)
                iota_s = lax.broadcasted_iota(jnp.int32, (T, S), 1)
                sc = jnp.where(iota_s < len_ref[b * A + a], sc, -1e9)
                m = jnp.max(sc, axis=1, keepdims=True)
                e = jnp.exp(sc - m)
                attn = e / jnp.sum(e, axis=1, keepdims=True)
                ctx = jnp.dot(attn, enc, preferred_element_type=jnp.float32)
                comb = ctx + dec_b
                ge = 1.0 / (1.0 + jnp.exp(-jnp.sum(comb * wgen_ref[...],
                                                   axis=1, keepdims=True)))
                ga = jnp.sum(comb * wagent_ref[...], axis=1, keepdims=True)
                attns.append(attn)
                ges.append(ge)
                gas.append(ga)
            m = jnp.maximum(jnp.maximum(gas[0], gas[1]), gas[2])
            es = [jnp.exp(g - m) for g in gas]
            den = es[0] + es[1] + es[2]
            iota_t = lax.broadcasted_iota(jnp.int32, (T, 1), 0)
            tgt = (iota_t < plen_ref[b]).astype(jnp.float32)
            aas = [e / den for e in es]
            gmix_ref[b * T:(b + 1) * T, :] = (
                aas[0] * ges[0] + aas[1] * ges[1] + aas[2] * ges[2]) * tgt
            for a in range(A):
                cw_ref[a, b * T:(b + 1) * T, :] = (
                    aas[a] * (1.0 - ges[a]) * tgt) * attns[a]

    return pl.pallas_call(
        body,
        out_shape=[
            jax.ShapeDtypeStruct((BT, H), jnp.float32),
            jax.ShapeDtypeStruct((BT, 1), jnp.float32),
            jax.ShapeDtypeStruct((A, BT, S), jnp.float32),
        ],
        in_specs=[pl.BlockSpec(memory_space=pltpu.VMEM)] * 7
        + [pl.BlockSpec(memory_space=pltpu.SMEM)] * 2,
        out_specs=[pl.BlockSpec(memory_space=pltpu.VMEM)] * 3,
    )(emb_art, emb_prev2d, W_enc, W_dec, W_attn, wgen_row, wagent_row,
      lengths, prev_len)


def _scatter_sc(cw, art_bas):
    mesh = plsc.VectorSubcoreMesh(core_axis_name="c", subcore_axis_name="s")

    @functools.partial(
        pl.kernel,
        out_type=jax.ShapeDtypeStruct((BT * ZP,), jnp.float32),
        mesh=mesh,
        compiler_params=pltpu.CompilerParams(needs_layout_passes=False),
        scratch_types=[
            pltpu.VMEM((A * S,), jnp.int32),
            pltpu.VMEM((A * S,), jnp.float32),
            pltpu.VMEM((ZP,), jnp.float32),
        ],
    )
    def k(cw_hbm, art_hbm, out_hbm, idx_v, val_v, rowbuf):
        wid = lax.axis_index("s") * NC + lax.axis_index("c")
        b = wid // (NW // B)
        pltpu.sync_copy(art_hbm.at[pl.ds(b * A * S, A * S)], idx_v)
        zero16 = jnp.zeros((16,), jnp.float32)
        for r in range(BT // NW):
            row = wid * (BT // NW) + r

            def zbody(i, carry):
                base = i * 128
                for k2 in range(8):
                    rowbuf[pl.ds(base + k2 * 16, 16)] = zero16
                return carry

            lax.fori_loop(0, ZP // 128, zbody, 0)
            for a in range(A):
                pltpu.sync_copy(
                    cw_hbm.at[pl.ds((a * BT + row) * S, S)],
                    val_v.at[pl.ds(a * S, S)])
            for c in range(A * S // 16):
                plsc.addupdate_scatter(
                    rowbuf,
                    [idx_v[pl.ds(c * 16, 16)]],
                    val_v[pl.ds(c * 16, 16)])
            pltpu.sync_copy(rowbuf, out_hbm.at[pl.ds(row * ZP, ZP)])

    return k(cw.reshape(-1), art_bas.reshape(-1))


def _vocab_tc(dec_bf, W_out_bf, gmix, scat):
    def body(dec_ref, wout_ref, gmix_ref, scat_ref, out_ref, e_scr, acc_ref):
        p = pl.program_id(0)
        j = pl.program_id(1)

        @pl.when(jnp.logical_and(p == 0, j == 0))
        def _():
            acc_ref[...] = jnp.zeros_like(acc_ref)

        @pl.when(p == 0)
        def _():
            l = jnp.dot(dec_ref[...], wout_ref[...],
                        preferred_element_type=jnp.float32)
            gcol = j * VT + lax.broadcasted_iota(jnp.int32, (BT, VT), 1)
            e = jnp.where(gcol < V, jnp.exp(l), 0.0)
            e_scr[:, pl.ds(j * VT, VT)] = e
            acc_ref[...] = acc_ref[...] + jnp.sum(e, axis=1, keepdims=True)

        @pl.when(p == 1)
        def _():
            scat_t = jnp.concatenate(
                [jnp.reshape(scat_ref[:, :, 0, jj, :], (BT, 128))
                 for jj in range(VT // 128)], axis=1)
            out_ref[...] = (gmix_ref[...] * e_scr[:, pl.ds(j * VT, VT)]
                            / acc_ref[...] + scat_t)

    return pl.pallas_call(
        body,
        grid=(2, NT),
        in_specs=[
            pl.BlockSpec((BT, H), lambda p, j: (0, 0)),
            pl.BlockSpec(
                (H, VT),
                lambda p, j: (0, jnp.minimum(j, (V - 1) // VT) * (1 - p))),
            pl.BlockSpec((BT, 1), lambda p, j: (0, 0)),
            pl.BlockSpec((BT // 8, 8, 1, VT // 128, 128),
                         lambda p, j: (0, 0, j * p, 0, 0)),
        ],
        out_specs=pl.BlockSpec((BT, VT), lambda p, j: (0, j * p)),
        out_shape=jax.ShapeDtypeStruct((BT, EXT), jnp.float32),
        scratch_shapes=[pltpu.VMEM((BT, NT * VT), jnp.float32),
                        pltpu.VMEM((BT, 1), jnp.float32)],
    )(dec_bf, W_out_bf, gmix, scat)


def kernel(article, article_length, prev_input, prev_input_length, table,
           W_enc, W_dec, W_attn, w_gen, w_agent, W_out):
    art_bas = article.transpose(1, 2, 0).astype(jnp.int32)
    idx_art = art_bas.reshape(-1)
    idx_prev = prev_input.transpose(1, 0).reshape(-1)
    idx_all = jnp.concatenate(
        [idx_art, idx_prev,
         jnp.zeros((NG - idx_art.size - idx_prev.size,), jnp.int32)])
    emb_all = _gather_sc(table, idx_all)
    emb_art = emb_all[:B * A * S].reshape(B, A, S, D)
    emb_prev2d = emb_all[B * A * S:B * A * S + BT]

    lengths = jnp.maximum(article_length, 1).astype(jnp.int32)
    prev_len = jnp.maximum(prev_input_length, 1).astype(jnp.int32)

    dec2d, gmix, cw = _attention_tc(
        emb_art, emb_prev2d, W_enc, W_dec, W_attn,
        w_gen.reshape(1, H), w_agent.reshape(1, H), lengths, prev_len)

    scat = _scatter_sc(cw, art_bas).reshape(BT // 8, 8, ZP // VT, VT // 128, 128)
    out = _vocab_tc(dec2d.astype(jnp.bfloat16), W_out.astype(jnp.bfloat16),
                    gmix, scat)
    return out.reshape(B, T, EXT)


# speedup vs baseline: 7.1579x; 1.3246x over previous
import functools

import jax
import jax.numpy as jnp
from jax import lax
from jax.experimental import pallas as pl
from jax.experimental.pallas import tpu as pltpu
from jax.experimental.pallas import tpu_sc as plsc

V = 50000
EXT = 51000
D = 128
H = 256
S = 400
B = 4
A = 3
T = 32
BT = B * T
VT = 2048
NT = -(-EXT // VT)
ZP = 51200

NC = 2
NS = 16
NW = NC * NS

NG = 5120
GPW = NG // NW


def _gather_sc(table, idx_all):
    mesh = plsc.VectorSubcoreMesh(core_axis_name="c", subcore_axis_name="s")

    @functools.partial(
        pl.kernel,
        out_type=jax.ShapeDtypeStruct((NG, D), jnp.float32),
        mesh=mesh,
        scratch_types=[
            pltpu.VMEM((GPW // 2,), jnp.int32),
            pltpu.VMEM((GPW // 2,), jnp.int32),
            pltpu.VMEM((GPW, D), jnp.float32),
            pltpu.SemaphoreType.DMA,
            pltpu.SemaphoreType.DMA,
        ],
    )
    def k(table_hbm, idx_hbm, out_hbm, idx_a, idx_b, rows_v, sem_a, sem_b):
        wid = lax.axis_index("s") * NC + lax.axis_index("c")
        base = wid * GPW
        half = GPW // 2
        pltpu.sync_copy(idx_hbm.at[pl.ds(base, half)], idx_a)
        pltpu.sync_copy(idx_hbm.at[pl.ds(base + half, half)], idx_b)
        c1 = pltpu.async_copy(table_hbm.at[idx_a], rows_v.at[pl.ds(0, half)], sem_a)
        c2 = pltpu.async_copy(table_hbm.at[idx_b], rows_v.at[pl.ds(half, half)], sem_b)
        c1.wait()
        c2.wait()
        pltpu.sync_copy(rows_v, out_hbm.at[pl.ds(base, GPW)])

    return k(table, idx_all)


def _attention_tc(emb_art, emb_prev2d, W_enc, W_dec, W_attn, wgen_row,
                  wagent_row, lengths, prev_len):
    def body(emb_art_ref, emb_prev_ref, wenc_ref, wdec_ref, wattn_ref,
             wgen_ref, wagent_ref, len_ref, plen_ref,
             dec_ref, gmix_ref, cw_ref):
        dec2d = jnp.tanh(jnp.dot(emb_prev_ref[...], wdec_ref[...],
                                 preferred_element_type=jnp.float32))
        proj = jnp.dot(dec2d, wattn_ref[...], preferred_element_type=jnp.float32)
        dec_ref[...] = dec2d
        for b in range(B):
            proj_b = proj[b * T:(b + 1) * T]
            dec_b = dec2d[b * T:(b + 1) * T]
            attns, ges, gas = [], [], []
            for a in range(A):
                enc = jnp.tanh(jnp.dot(emb_art_ref[b, a], wenc_ref[...],
                                       preferred_element_type=jnp.float32))
                sc = lax.dot_general(proj_b, enc, (((1,), (1,)), ((), ())),
                                     preferred_element_type=jnp.float32)
                iota_s = lax.broadcasted_iota(jnp.int32, (T, S), 1)
                sc = jnp.where(iota_s < len_ref[b * A + a], sc, -1e9)
                m = jnp.max(sc, axis=1, keepdims=True)
                e = jnp.exp(sc - m)
                attn = e / jnp.sum(e, axis=1, keepdims=True)
                ctx = jnp.dot(attn, enc, preferred_element_type=jnp.float32)
                comb = ctx + dec_b
                ge = 1.0 / (1.0 + jnp.exp(-jnp.sum(comb * wgen_ref[...],
                                                   axis=1, keepdims=True)))
                ga = jnp.sum(comb * wagent_ref[...], axis=1, keepdims=True)
                attns.append(attn)
                ges.append(ge)
                gas.append(ga)
            m = jnp.maximum(jnp.maximum(gas[0], gas[1]), gas[2])
            es = [jnp.exp(g - m) for g in gas]
            den = es[0] + es[1] + es[2]
            iota_t = lax.broadcasted_iota(jnp.int32, (T, 1), 0)
            tgt = (iota_t < plen_ref[b]).astype(jnp.float32)
            aas = [e / den for e in es]
            gmix_ref[b * T:(b + 1) * T, :] = (
                aas[0] * ges[0] + aas[1] * ges[1] + aas[2] * ges[2]) * tgt
            for a in range(A):
                cw_ref[a, b * T:(b + 1) * T, :] = (
                    aas[a] * (1.0 - ges[a]) * tgt) * attns[a]

    return pl.pallas_call(
        body,
        out_shape=[
            jax.ShapeDtypeStruct((BT, H), jnp.float32),
            jax.ShapeDtypeStruct((BT, 1), jnp.float32),
            jax.ShapeDtypeStruct((A, BT, S), jnp.float32),
        ],
        in_specs=[pl.BlockSpec(memory_space=pltpu.VMEM)] * 7
        + [pl.BlockSpec(memory_space=pltpu.SMEM)] * 2,
        out_specs=[pl.BlockSpec(memory_space=pltpu.VMEM)] * 3,
    )(emb_art, emb_prev2d, W_enc, W_dec, W_attn, wgen_row, wagent_row,
      lengths, prev_len)


def _scatter_sc(cw, art_bas):
    mesh = plsc.VectorSubcoreMesh(core_axis_name="c", subcore_axis_name="s")

    @functools.partial(
        pl.kernel,
        out_type=jax.ShapeDtypeStruct((BT * ZP,), jnp.float32),
        mesh=mesh,
        compiler_params=pltpu.CompilerParams(needs_layout_passes=False),
        scratch_types=[
            pltpu.VMEM((A * S,), jnp.int32),
            pltpu.VMEM((A * S,), jnp.float32),
            pltpu.VMEM((ZP,), jnp.float32),
        ],
    )
    def k(cw_hbm, art_hbm, out_hbm, idx_v, val_v, rowbuf):
        wid = lax.axis_index("s") * NC + lax.axis_index("c")
        b = wid // (NW // B)
        pltpu.sync_copy(art_hbm.at[pl.ds(b * A * S, A * S)], idx_v)
        zero16 = jnp.zeros((16,), jnp.float32)
        for r in range(BT // NW):
            row = wid * (BT // NW) + r

            def zbody(i, carry):
                base = i * 128
                for k2 in range(8):
                    rowbuf[pl.ds(base + k2 * 16, 16)] = zero16
                return carry

            lax.fori_loop(0, ZP // 128, zbody, 0)
            for a in range(A):
                pltpu.sync_copy(
                    cw_hbm.at[pl.ds((a * BT + row) * S, S)],
                    val_v.at[pl.ds(a * S, S)])
            for c in range(A * S // 16):
                plsc.addupdate_scatter(
                    rowbuf,
                    [idx_v[pl.ds(c * 16, 16)]],
                    val_v[pl.ds(c * 16, 16)])
            pltpu.sync_copy(rowbuf, out_hbm.at[pl.ds(row * ZP, ZP)])

    return k(cw.reshape(-1), art_bas.reshape(-1))


def _vocab_tc(dec_bf, W_out_T_bf, gmix, scat):
    def body(dec_ref, wout_ref, gmix_ref, scat_ref, out_ref, e_scr,
             acc_ref, inv_ref):
        p = pl.program_id(0)
        j = pl.program_id(1)

        @pl.when(jnp.logical_and(p == 0, j == 0))
        def _():
            acc_ref[...] = jnp.zeros_like(acc_ref)

        @pl.when(p == 0)
        def _():
            l = lax.dot_general(dec_ref[...], wout_ref[...],
                                (((1,), (1,)), ((), ())),
                                preferred_element_type=jnp.float32)
            gcol = j * VT + lax.broadcasted_iota(jnp.int32, (BT, VT), 1)
            e = jnp.where(gcol < V, jnp.exp(l), 0.0)
            e_scr[:, pl.ds(j * VT, VT)] = e
            acc_ref[...] = acc_ref[...] + jnp.sum(e, axis=1, keepdims=True)

        @pl.when(jnp.logical_and(p == 1, j == 0))
        def _():
            inv_ref[...] = gmix_ref[...] / acc_ref[...]

        @pl.when(p == 1)
        def _():
            scat_t = jnp.concatenate(
                [jnp.reshape(scat_ref[:, :, 0, jj, :], (BT, 128))
                 for jj in range(VT // 128)], axis=1)
            out_ref[...] = (inv_ref[...] * e_scr[:, pl.ds(j * VT, VT)]
                            + scat_t)

    return pl.pallas_call(
        body,
        grid=(2, NT),
        in_specs=[
            pl.BlockSpec((BT, H), lambda p, j: (0, 0)),
            pl.BlockSpec(
                (VT, H),
                lambda p, j: (jnp.minimum(j, (V - 1) // VT) * (1 - p), 0)),
            pl.BlockSpec((BT, 1), lambda p, j: (0, 0)),
            pl.BlockSpec((BT // 8, 8, 1, VT // 128, 128),
                         lambda p, j: (0, 0, j * p, 0, 0)),
        ],
        out_specs=pl.BlockSpec((BT, VT), lambda p, j: (0, j * p)),
        out_shape=jax.ShapeDtypeStruct((BT, EXT), jnp.float32),
        scratch_shapes=[pltpu.VMEM((BT, NT * VT), jnp.float32),
                        pltpu.VMEM((BT, 1), jnp.float32),
                        pltpu.VMEM((BT, 1), jnp.float32)],
    )(dec_bf, W_out_T_bf, gmix, scat)


def kernel(article, article_length, prev_input, prev_input_length, table,
           W_enc, W_dec, W_attn, w_gen, w_agent, W_out):
    art_bas = article.transpose(1, 2, 0).astype(jnp.int32)
    idx_art = art_bas.reshape(-1)
    idx_prev = prev_input.transpose(1, 0).reshape(-1)
    idx_all = jnp.concatenate(
        [idx_art, idx_prev,
         jnp.zeros((NG - idx_art.size - idx_prev.size,), jnp.int32)])
    emb_all = _gather_sc(table, idx_all)
    emb_art = emb_all[:B * A * S].reshape(B, A, S, D)
    emb_prev2d = emb_all[B * A * S:B * A * S + BT]

    lengths = jnp.maximum(article_length, 1).astype(jnp.int32)
    prev_len = jnp.maximum(prev_input_length, 1).astype(jnp.int32)

    dec2d, gmix, cw = _attention_tc(
        emb_art, emb_prev2d, W_enc, W_dec, W_attn,
        w_gen.reshape(1, H), w_agent.reshape(1, H), lengths, prev_len)

    scat = _scatter_sc(cw, art_bas).reshape(BT // 8, 8, ZP // VT, VT // 128, 128)
    out = _vocab_tc(dec2d.astype(jnp.bfloat16), W_out.astype(jnp.bfloat16).T,
                    gmix, scat)
    return out.reshape(B, T, EXT)


# speedup vs baseline: 7.6445x; 1.0680x over previous
import functools

import jax
import jax.numpy as jnp
from jax import lax
from jax.experimental import pallas as pl
from jax.experimental.pallas import tpu as pltpu
from jax.experimental.pallas import tpu_sc as plsc

V = 50000
EXT = 51000
D = 128
H = 256
S = 400
B = 4
A = 3
T = 32
BT = B * T
VT = 2048
NT = -(-EXT // VT)
ZP = 51200

NC = 2
NS = 16
NW = NC * NS

NG = 5120
GPW = NG // NW


def _gather_sc(table, idx_all):
    mesh = plsc.VectorSubcoreMesh(core_axis_name="c", subcore_axis_name="s")

    @functools.partial(
        pl.kernel,
        out_type=jax.ShapeDtypeStruct((NG, D), jnp.float32),
        mesh=mesh,
        scratch_types=[
            pltpu.VMEM((GPW // 2,), jnp.int32),
            pltpu.VMEM((GPW // 2,), jnp.int32),
            pltpu.VMEM((GPW, D), jnp.float32),
            pltpu.SemaphoreType.DMA,
            pltpu.SemaphoreType.DMA,
        ],
    )
    def k(table_hbm, idx_hbm, out_hbm, idx_a, idx_b, rows_v, sem_a, sem_b):
        wid = lax.axis_index("s") * NC + lax.axis_index("c")
        base = wid * GPW
        half = GPW // 2
        pltpu.sync_copy(idx_hbm.at[pl.ds(base, half)], idx_a)
        pltpu.sync_copy(idx_hbm.at[pl.ds(base + half, half)], idx_b)
        c1 = pltpu.async_copy(table_hbm.at[idx_a], rows_v.at[pl.ds(0, half)], sem_a)
        c2 = pltpu.async_copy(table_hbm.at[idx_b], rows_v.at[pl.ds(half, half)], sem_b)
        c1.wait()
        c2.wait()
        pltpu.sync_copy(rows_v, out_hbm.at[pl.ds(base, GPW)])

    return k(table, idx_all)


def _attention_tc(emb_art, emb_prev2d, W_enc, W_dec, W_attn, wgen_row,
                  wagent_row, lengths, prev_len):
    def body(emb_art_ref, emb_prev_ref, wenc_ref, wdec_ref, wattn_ref,
             wgen_ref, wagent_ref, len_ref, plen_ref,
             dec_ref, gmix_ref, cw_ref):
        dec2d = jnp.tanh(jnp.dot(emb_prev_ref[...], wdec_ref[...],
                                 preferred_element_type=jnp.float32))
        proj = jnp.dot(dec2d, wattn_ref[...], preferred_element_type=jnp.float32)
        dec_ref[...] = dec2d
        for b in range(B):
            proj_b = proj[b * T:(b + 1) * T]
            dec_b = dec2d[b * T:(b + 1) * T]
            attns, ges, gas = [], [], []
            for a in range(A):
                enc = jnp.tanh(jnp.dot(emb_art_ref[b, a], wenc_ref[...],
                                       preferred_element_type=jnp.float32))
                sc = lax.dot_general(proj_b, enc, (((1,), (1,)), ((), ())),
                                     preferred_element_type=jnp.float32)
                iota_s = lax.broadcasted_iota(jnp.int32, (T, S), 1)
                sc = jnp.where(iota_s < len_ref[b * A + a], sc, -1e9)
                m = jnp.max(sc, axis=1, keepdims=True)
                e = jnp.exp(sc - m)
                attn = e / jnp.sum(e, axis=1, keepdims=True)
                ctx = jnp.dot(attn, enc, preferred_element_type=jnp.float32)
                comb = ctx + dec_b
                ge = 1.0 / (1.0 + jnp.exp(-jnp.sum(comb * wgen_ref[...],
                                                   axis=1, keepdims=True)))
                ga = jnp.sum(comb * wagent_ref[...], axis=1, keepdims=True)
                attns.append(attn)
                ges.append(ge)
                gas.append(ga)
            m = jnp.maximum(jnp.maximum(gas[0], gas[1]), gas[2])
            es = [jnp.exp(g - m) for g in gas]
            den = es[0] + es[1] + es[2]
            iota_t = lax.broadcasted_iota(jnp.int32, (T, 1), 0)
            tgt = (iota_t < plen_ref[b]).astype(jnp.float32)
            aas = [e / den for e in es]
            gmix_ref[b * T:(b + 1) * T, :] = (
                aas[0] * ges[0] + aas[1] * ges[1] + aas[2] * ges[2]) * tgt
            for a in range(A):
                cw_ref[a, b * T:(b + 1) * T, :] = (
                    aas[a] * (1.0 - ges[a]) * tgt) * attns[a]

    return pl.pallas_call(
        body,
        out_shape=[
            jax.ShapeDtypeStruct((BT, H), jnp.float32),
            jax.ShapeDtypeStruct((BT, 1), jnp.float32),
            jax.ShapeDtypeStruct((A, BT, S), jnp.float32),
        ],
        in_specs=[pl.BlockSpec(memory_space=pltpu.VMEM)] * 7
        + [pl.BlockSpec(memory_space=pltpu.SMEM)] * 2,
        out_specs=[pl.BlockSpec(memory_space=pltpu.VMEM)] * 3,
    )(emb_art, emb_prev2d, W_enc, W_dec, W_attn, wgen_row, wagent_row,
      lengths, prev_len)


def _scatter_sc(cw, art_bas):
    mesh = plsc.VectorSubcoreMesh(core_axis_name="c", subcore_axis_name="s")

    rpw = BT // NW

    @functools.partial(
        pl.kernel,
        out_type=jax.ShapeDtypeStruct((BT * ZP,), jnp.float32),
        mesh=mesh,
        compiler_params=pltpu.CompilerParams(needs_layout_passes=False),
        scratch_types=[
            pltpu.VMEM((A * S,), jnp.int32),
            pltpu.VMEM((rpw * A * S,), jnp.float32),
            pltpu.VMEM((ZP,), jnp.float32),
            pltpu.VMEM((ZP,), jnp.float32),
            pltpu.SemaphoreType.DMA,
            pltpu.SemaphoreType.DMA,
            pltpu.SemaphoreType.DMA,
        ],
    )
    def k(cw_hbm, art_hbm, out_hbm, idx_v, val_v, buf0, buf1,
          sem0, sem1, semv):
        wid = lax.axis_index("s") * NC + lax.axis_index("c")
        b = wid // (NW // B)
        pltpu.sync_copy(art_hbm.at[pl.ds(b * A * S, A * S)], idx_v)
        vcopies = []
        for r in range(rpw):
            row = wid * rpw + r
            for a in range(A):
                vcopies.append(pltpu.async_copy(
                    cw_hbm.at[pl.ds((a * BT + row) * S, S)],
                    val_v.at[pl.ds((r * A + a) * S, S)], semv))
        zero16 = jnp.zeros((16,), jnp.float32)

        def zbody(i, carry):
            base = i * 128
            for k2 in range(8):
                buf0[pl.ds(base + k2 * 16, 16)] = zero16
                buf1[pl.ds(base + k2 * 16, 16)] = zero16
            return carry

        lax.fori_loop(0, ZP // 128, zbody, 0)
        for c in vcopies:
            c.wait()
        bufs = [buf0, buf1]
        sems = [sem0, sem1]
        dmas = [None, None]
        for r in range(rpw):
            row = wid * rpw + r
            buf = bufs[r % 2]
            if dmas[r % 2] is not None:
                dmas[r % 2].wait()
                for c in range(A * S // 16):
                    plsc.store_scatter(buf, [idx_v[pl.ds(c * 16, 16)]], zero16)
            for c in range(A * S // 16):
                plsc.addupdate_scatter(
                    buf,
                    [idx_v[pl.ds(c * 16, 16)]],
                    val_v[pl.ds((r * A * S) + c * 16, 16)])
            dmas[r % 2] = pltpu.async_copy(
                buf, out_hbm.at[pl.ds(row * ZP, ZP)], sems[r % 2])
        dmas[0].wait()
        dmas[1].wait()

    return k(cw.reshape(-1), art_bas.reshape(-1))


def _vocab_tc(dec_bf, W_out_T_bf, gmix, scat):
    def body(dec_ref, wout_ref, gmix_ref, scat_ref, out_ref, e_scr,
             acc_ref, inv_ref):
        p = pl.program_id(0)
        j = pl.program_id(1)

        @pl.when(jnp.logical_and(p == 0, j == 0))
        def _():
            acc_ref[...] = jnp.zeros_like(acc_ref)

        @pl.when(p == 0)
        def _():
            l = lax.dot_general(dec_ref[...], wout_ref[...],
                                (((1,), (1,)), ((), ())),
                                preferred_element_type=jnp.float32)
            gcol = j * VT + lax.broadcasted_iota(jnp.int32, (BT, VT), 1)
            e = jnp.where(gcol < V, jnp.exp(l), 0.0)
            e_scr[:, pl.ds(j * VT, VT)] = e
            acc_ref[...] = acc_ref[...] + jnp.sum(e, axis=1, keepdims=True)

        @pl.when(jnp.logical_and(p == 1, j == 0))
        def _():
            inv_ref[...] = gmix_ref[...] / acc_ref[...]

        @pl.when(p == 1)
        def _():
            scat_t = jnp.concatenate(
                [jnp.reshape(scat_ref[:, :, 0, jj, :], (BT, 128))
                 for jj in range(VT // 128)], axis=1)
            out_ref[...] = (inv_ref[...] * e_scr[:, pl.ds(j * VT, VT)]
                            + scat_t)

    return pl.pallas_call(
        body,
        grid=(2, NT),
        in_specs=[
            pl.BlockSpec((BT, H), lambda p, j: (0, 0)),
            pl.BlockSpec(
                (VT, H),
                lambda p, j: (jnp.minimum(j, (V - 1) // VT) * (1 - p), 0)),
            pl.BlockSpec((BT, 1), lambda p, j: (0, 0)),
            pl.BlockSpec((BT // 8, 8, 1, VT // 128, 128),
                         lambda p, j: (0, 0, j * p, 0, 0)),
        ],
        out_specs=pl.BlockSpec((BT, VT), lambda p, j: (0, j * p)),
        out_shape=jax.ShapeDtypeStruct((BT, EXT), jnp.float32),
        scratch_shapes=[pltpu.VMEM((BT, NT * VT), jnp.float32),
                        pltpu.VMEM((BT, 1), jnp.float32),
                        pltpu.VMEM((BT, 1), jnp.float32)],
    )(dec_bf, W_out_T_bf, gmix, scat)


def kernel(article, article_length, prev_input, prev_input_length, table,
           W_enc, W_dec, W_attn, w_gen, w_agent, W_out):
    art_bas = article.transpose(1, 2, 0).astype(jnp.int32)
    idx_art = art_bas.reshape(-1)
    idx_prev = prev_input.transpose(1, 0).reshape(-1)
    idx_all = jnp.concatenate(
        [idx_art, idx_prev,
         jnp.zeros((NG - idx_art.size - idx_prev.size,), jnp.int32)])
    emb_all = _gather_sc(table, idx_all)
    emb_art = emb_all[:B * A * S].reshape(B, A, S, D)
    emb_prev2d = emb_all[B * A * S:B * A * S + BT]

    lengths = jnp.maximum(article_length, 1).astype(jnp.int32)
    prev_len = jnp.maximum(prev_input_length, 1).astype(jnp.int32)

    dec2d, gmix, cw = _attention_tc(
        emb_art, emb_prev2d, W_enc, W_dec, W_attn,
        w_gen.reshape(1, H), w_agent.reshape(1, H), lengths, prev_len)

    scat = _scatter_sc(cw, art_bas).reshape(BT // 8, 8, ZP // VT, VT // 128, 128)
    out = _vocab_tc(dec2d.astype(jnp.bfloat16), W_out.astype(jnp.bfloat16).T,
                    gmix, scat)
    return out.reshape(B, T, EXT)


# speedup vs baseline: 9.4657x; 1.2382x over previous
import functools

import jax
import jax.numpy as jnp
from jax import lax
from jax.experimental import pallas as pl
from jax.experimental.pallas import tpu as pltpu
from jax.experimental.pallas import tpu_sc as plsc

V = 50000
EXT = 51000
D = 128
H = 256
S = 400
B = 4
A = 3
T = 32
BT = B * T
VT = 2048
NT = -(-EXT // VT)
ZP = 51200

NC = 2
NS = 16
NW = NC * NS

NGA = B * A * S
GPW = NGA // 30
NGP = 160


def _gather_sc(table, idx_art, idx_prev_pad):
    mesh = plsc.VectorSubcoreMesh(core_axis_name="c", subcore_axis_name="s")
    half = GPW // 2

    @functools.partial(
        pl.kernel,
        out_type=[jax.ShapeDtypeStruct((NGA, D), jnp.float32),
                  jax.ShapeDtypeStruct((NGP, D), jnp.float32)],
        mesh=mesh,
        scratch_types=[
            pltpu.VMEM((half,), jnp.int32),
            pltpu.VMEM((half,), jnp.int32),
            pltpu.VMEM((GPW, D), jnp.float32),
            pltpu.SemaphoreType.DMA,
            pltpu.SemaphoreType.DMA,
        ],
    )
    def k(table_hbm, ia_hbm, ip_hbm, oa_hbm, op_hbm,
          idx_a, idx_b, rows_v, sem_a, sem_b):
        wid = lax.axis_index("s") * NC + lax.axis_index("c")

        @pl.when(wid < 30)
        def _():
            base = wid * GPW
            pltpu.sync_copy(ia_hbm.at[pl.ds(base, half)], idx_a)
            c1 = pltpu.async_copy(table_hbm.at[idx_a],
                                  rows_v.at[pl.ds(0, half)], sem_a)
            pltpu.sync_copy(ia_hbm.at[pl.ds(base + half, half)], idx_b)
            c2 = pltpu.async_copy(table_hbm.at[idx_b],
                                  rows_v.at[pl.ds(half, half)], sem_b)
            c1.wait()
            c2.wait()
            pltpu.sync_copy(rows_v, oa_hbm.at[pl.ds(base, GPW)])

        @pl.when(wid >= 30)
        def _():
            base = (wid - 30) * half
            pltpu.sync_copy(ip_hbm.at[pl.ds(base, half)], idx_a)
            pltpu.async_copy(table_hbm.at[idx_a],
                             rows_v.at[pl.ds(0, half)], sem_a).wait()
            pltpu.sync_copy(rows_v.at[pl.ds(0, half)],
                            op_hbm.at[pl.ds(base, half)])

    return k(table, idx_art, idx_prev_pad)


def _attention_tc(emb_art, emb_prev2d, W_enc, W_dec, W_attn, wgen_row,
                  wagent_row, lengths, prev_len):
    def body(emb_art_ref, emb_prev_ref, wenc_ref, wdec_ref, wattn_ref,
             wgen_ref, wagent_ref, len_ref, plen_ref,
             dec_ref, gmix_ref, cw_ref):
        dec2d = jnp.tanh(jnp.dot(emb_prev_ref[0:BT, :], wdec_ref[...],
                                 preferred_element_type=jnp.float32))
        proj = jnp.dot(dec2d, wattn_ref[...], preferred_element_type=jnp.float32)
        dec_ref[...] = dec2d
        for b in range(B):
            proj_b = proj[b * T:(b + 1) * T]
            dec_b = dec2d[b * T:(b + 1) * T]
            attns, ges, gas = [], [], []
            for a in range(A):
                enc = jnp.tanh(jnp.dot(emb_art_ref[b, a], wenc_ref[...],
                                       preferred_element_type=jnp.float32))
                sc = lax.dot_general(proj_b, enc, (((1,), (1,)), ((), ())),
                                     preferred_element_type=jnp.float32)
                iota_s = lax.broadcasted_iota(jnp.int32, (T, S), 1)
                sc = jnp.where(iota_s < len_ref[b * A + a], sc, -1e9)
                m = jnp.max(sc, axis=1, keepdims=True)
                e = jnp.exp(sc - m)
                attn = e / jnp.sum(e, axis=1, keepdims=True)
                ctx = jnp.dot(attn, enc, preferred_element_type=jnp.float32)
                comb = ctx + dec_b
                ge = 1.0 / (1.0 + jnp.exp(-jnp.sum(comb * wgen_ref[...],
                                                   axis=1, keepdims=True)))
                ga = jnp.sum(comb * wagent_ref[...], axis=1, keepdims=True)
                attns.append(attn)
                ges.append(ge)
                gas.append(ga)
            m = jnp.maximum(jnp.maximum(gas[0], gas[1]), gas[2])
            es = [jnp.exp(g - m) for g in gas]
            den = es[0] + es[1] + es[2]
            iota_t = lax.broadcasted_iota(jnp.int32, (T, 1), 0)
            tgt = (iota_t < plen_ref[b]).astype(jnp.float32)
            aas = [e / den for e in es]
            gmix_ref[b * T:(b + 1) * T, :] = (
                aas[0] * ges[0] + aas[1] * ges[1] + aas[2] * ges[2]) * tgt
            for a in range(A):
                cw_ref[a, b * T:(b + 1) * T, :] = (
                    aas[a] * (1.0 - ges[a]) * tgt) * attns[a]

    return pl.pallas_call(
        body,
        out_shape=[
            jax.ShapeDtypeStruct((BT, H), jnp.float32),
            jax.ShapeDtypeStruct((BT, 1), jnp.float32),
            jax.ShapeDtypeStruct((A, BT, S), jnp.float32),
        ],
        in_specs=[pl.BlockSpec(memory_space=pltpu.VMEM)] * 7
        + [pl.BlockSpec(memory_space=pltpu.SMEM)] * 2,
        out_specs=[pl.BlockSpec(memory_space=pltpu.VMEM)] * 3,
    )(emb_art, emb_prev2d, W_enc, W_dec, W_attn, wgen_row, wagent_row,
      lengths, prev_len)


def _scatter_sc(cw, art_bas):
    mesh = plsc.VectorSubcoreMesh(core_axis_name="c", subcore_axis_name="s")

    rpw = BT // NW

    @functools.partial(
        pl.kernel,
        out_type=jax.ShapeDtypeStruct((BT * ZP,), jnp.float32),
        mesh=mesh,
        compiler_params=pltpu.CompilerParams(needs_layout_passes=False),
        scratch_types=[
            pltpu.VMEM((A * S,), jnp.int32),
            pltpu.VMEM((rpw * A * S,), jnp.float32),
            pltpu.VMEM((ZP,), jnp.float32),
            pltpu.VMEM((ZP,), jnp.float32),
            pltpu.SemaphoreType.DMA,
            pltpu.SemaphoreType.DMA,
            pltpu.SemaphoreType.DMA,
        ],
    )
    def k(cw_hbm, art_hbm, out_hbm, idx_v, val_v, buf0, buf1,
          sem0, sem1, semv):
        wid = lax.axis_index("s") * NC + lax.axis_index("c")
        b = wid // (NW // B)
        pltpu.sync_copy(art_hbm.at[pl.ds(b * A * S, A * S)], idx_v)
        vcopies = []
        for r in range(rpw):
            row = wid * rpw + r
            for a in range(A):
                vcopies.append(pltpu.async_copy(
                    cw_hbm.at[pl.ds((a * BT + row) * S, S)],
                    val_v.at[pl.ds((r * A + a) * S, S)], semv))
        zero16 = jnp.zeros((16,), jnp.float32)

        def zbody(i, carry):
            base = i * 128
            for k2 in range(8):
                buf0[pl.ds(base + k2 * 16, 16)] = zero16
                buf1[pl.ds(base + k2 * 16, 16)] = zero16
            return carry

        lax.fori_loop(0, ZP // 128, zbody, 0)
        for c in vcopies:
            c.wait()
        bufs = [buf0, buf1]
        sems = [sem0, sem1]
        dmas = [None, None]
        for r in range(rpw):
            row = wid * rpw + r
            buf = bufs[r % 2]
            if dmas[r % 2] is not None:
                dmas[r % 2].wait()
                for c in range(A * S // 16):
                    plsc.store_scatter(buf, [idx_v[pl.ds(c * 16, 16)]], zero16)
            for c in range(A * S // 16):
                plsc.addupdate_scatter(
                    buf,
                    [idx_v[pl.ds(c * 16, 16)]],
                    val_v[pl.ds((r * A * S) + c * 16, 16)])
            dmas[r % 2] = pltpu.async_copy(
                buf, out_hbm.at[pl.ds(row * ZP, ZP)], sems[r % 2])
        dmas[0].wait()
        dmas[1].wait()

    return k(cw.reshape(-1), art_bas.reshape(-1))


def _vocab_p0_tc(dec2d, W_out_T):
    def body(dec_ref, wout_ref, e_ref, acc_ref):
        j = pl.program_id(0)

        @pl.when(j == 0)
        def _():
            acc_ref[...] = jnp.zeros_like(acc_ref)

        l = lax.dot_general(dec_ref[...], wout_ref[...],
                            (((1,), (1,)), ((), ())),
                            preferred_element_type=jnp.float32)
        gcol = j * VT + lax.broadcasted_iota(jnp.int32, (BT, VT), 1)
        e = jnp.where(gcol < V, jnp.exp(l), 0.0)
        e_ref[...] = e.astype(jnp.bfloat16)
        acc_ref[...] = acc_ref[...] + jnp.sum(e, axis=1, keepdims=True)

    return pl.pallas_call(
        body,
        grid=(NT,),
        in_specs=[
            pl.BlockSpec((BT, H), lambda j: (0, 0)),
            pl.BlockSpec((VT, H), lambda j: (jnp.minimum(j, (V - 1) // VT), 0)),
        ],
        out_specs=[pl.BlockSpec((BT, VT), lambda j: (0, j)),
                   pl.BlockSpec((BT, 1), lambda j: (0, 0))],
        out_shape=[jax.ShapeDtypeStruct((BT, NT * VT), jnp.bfloat16),
                   jax.ShapeDtypeStruct((BT, 1), jnp.float32)],
    )(dec2d, W_out_T)


def _vocab_p1_tc(e_bf, acc, gmix, scat):
    def body(e_ref, acc_ref, gmix_ref, scat_ref, out_ref, inv_ref):
        j = pl.program_id(0)

        @pl.when(j == 0)
        def _():
            inv_ref[...] = gmix_ref[...] / acc_ref[...]

        scat_t = jnp.concatenate(
            [jnp.reshape(scat_ref[:, :, 0, jj, :], (BT, 128))
             for jj in range(VT // 128)], axis=1)
        out_ref[...] = (inv_ref[...] * e_ref[...].astype(jnp.float32)
                        + scat_t)

    return pl.pallas_call(
        body,
        grid=(NT,),
        in_specs=[
            pl.BlockSpec((BT, VT), lambda j: (0, j)),
            pl.BlockSpec((BT, 1), lambda j: (0, 0)),
            pl.BlockSpec((BT, 1), lambda j: (0, 0)),
            pl.BlockSpec((BT // 8, 8, 1, VT // 128, 128),
                         lambda j: (0, 0, j, 0, 0)),
        ],
        out_specs=pl.BlockSpec((BT, VT), lambda j: (0, j)),
        out_shape=jax.ShapeDtypeStruct((BT, EXT), jnp.float32),
        scratch_shapes=[pltpu.VMEM((BT, 1), jnp.float32)],
    )(e_bf, acc, gmix, scat)


def kernel(article, article_length, prev_input, prev_input_length, table,
           W_enc, W_dec, W_attn, w_gen, w_agent, W_out):
    art_bas = article.transpose(1, 2, 0).astype(jnp.int32)
    idx_art = art_bas.reshape(-1)
    idx_prev = jnp.concatenate(
        [prev_input.transpose(1, 0).reshape(-1),
         jnp.zeros((NGP - BT,), jnp.int32)])
    emb_art_2d, emb_prev_pad = _gather_sc(table, idx_art, idx_prev)
    emb_art = emb_art_2d.reshape(B, A, S, D)

    lengths = jnp.maximum(article_length, 1).astype(jnp.int32)
    prev_len = jnp.maximum(prev_input_length, 1).astype(jnp.int32)

    dec2d, gmix, cw = _attention_tc(
        emb_art, emb_prev_pad, W_enc, W_dec, W_attn,
        w_gen.reshape(1, H), w_agent.reshape(1, H), lengths, prev_len)

    scat = _scatter_sc(cw, art_bas).reshape(BT // 8, 8, ZP // VT, VT // 128, 128)
    e_bf, acc = _vocab_p0_tc(dec2d, W_out.T)
    out = _vocab_p1_tc(e_bf, acc, gmix, scat)
    return out.reshape(B, T, EXT)
